# baseline (speedup 1.0000x reference)
import functools
import math

import jax
import jax.numpy as jnp
from jax import lax
from jax.experimental import pallas as pl
from jax.experimental.pallas import tpu as pltpu
from jax.experimental.pallas import tpu_sc as plsc

_N = 10000
_NHID = 128
_NCLASS = 40
_L = 8
_ALPHA = 0.1
_THETA = 0.5
_EPS = 1e-5

_NC = 2
_NS = 16
_NW = _NC * _NS
_CH = 80
_CPW = 128
_EPAD = _NW * _CPW * _CH
_ND = 10112
_RPT = _ND // _NS
_ZCH = (80, 80, 80, 80, 80, 80, 80, 72)
_CPG = 8
_NBUF = 4
_DN = 10240

_BLK = 400
_GRID = _N // _BLK


def _sc_deg(dst3, out, dst_v, ones_v, zbuf, sem, deg_sp):
    cid = lax.axis_index("c")
    sid = lax.axis_index("s")
    wid = cid * _NS + sid
    for c in range(_CH // 16):
        ones_v[pl.ds(c * 16, 16)] = jnp.ones((16,), jnp.float32)

    @pl.when(sid == 0)
    def _zero():
        def zrow(r, carry):
            zbuf[pl.ds(r * 16, 16)] = jnp.zeros((16,), jnp.float32)
            return carry
        lax.fori_loop(0, _DN // 16, zrow, 0)
        pltpu.sync_copy(zbuf, deg_sp)

    pltpu.sync_copy(dst3.at[wid], dst_v)
    plsc.subcore_barrier()

    def group(g, carry):
        for j in range(_CPG):
            pltpu.async_copy(ones_v, deg_sp.at[dst_v.at[g * _CPG + j]],
                             sem, add=True)
        for j in range(_CPG):
            pltpu.make_async_copy(
                ones_v, deg_sp.at[dst_v.at[g * _CPG + j]], sem).wait()
        return carry
    lax.fori_loop(0, _CPW // _CPG, group, 0)
    plsc.subcore_barrier()

    @pl.when(sid == 0)
    def _copy_out():
        pltpu.sync_copy(deg_sp, out.at[cid])


def _sc_prop(hs, src3, dst3, out, src_v, dst_v,
             rows0, rows1, rows2, rows3,
             gs0, gs1, gs2, gs3, ss0, ss1, ss2, ss3, agg):
    cid = lax.axis_index("c")
    sid = lax.axis_index("s")
    wid = cid * _NS + sid

    def zrow(r, carry):
        for c in range(_NHID // 16):
            rows0[r, pl.ds(c * 16, 16)] = jnp.zeros((16,), jnp.float32)
        return carry
    lax.fori_loop(0, _CH, zrow, 0)
    base = sid * _RPT
    off = 0
    for zc in _ZCH:
        pltpu.sync_copy(rows0.at[pl.ds(0, zc)],
                        agg.at[pl.ds(base + off, zc)])
        off += zc
    plsc.subcore_barrier()

    rows = (rows0, rows1, rows2, rows3)
    gs = (gs0, gs1, gs2, gs3)
    ss = (ss0, ss1, ss2, ss3)

    def group(g, carry):
        goff = pl.multiple_of(g * _CPG, _CPG)
        pltpu.sync_copy(src3.at[wid, pl.ds(goff, _CPG)], src_v)
        pltpu.sync_copy(dst3.at[wid, pl.ds(goff, _CPG)], dst_v)
        gat = {0: pltpu.async_copy(hs.at[src_v.at[0]], rows[0], gs[0])}
        sca = {}
        for j in range(_CPG):
            if j + 1 < _CPG:
                if j - (_NBUF - 1) >= 0:
                    sca[j - (_NBUF - 1)].wait()
                gat[j + 1] = pltpu.async_copy(
                    hs.at[src_v.at[j + 1]], rows[(j + 1) % _NBUF],
                    gs[(j + 1) % _NBUF])
            gat[j].wait()
            sca[j] = pltpu.async_copy(
                rows[j % _NBUF], agg.at[dst_v.at[j]], ss[j % _NBUF],
                add=True)
        for j in range(_CPG - _NBUF, _CPG):
            sca[j].wait()
        return carry
    lax.fori_loop(0, _CPW // _CPG, group, 0)
    plsc.subcore_barrier()
    pltpu.sync_copy(agg.at[pl.ds(base, _RPT)], out.at[cid, pl.ds(base, _RPT)])


def _tc_in(x_ref, w_ref, b_ref, g_ref, bt_ref, d0_ref, d1_ref,
           h0_ref, hs_ref, dinv_ref):
    z = jnp.dot(x_ref[...], w_ref[...], preferred_element_type=jnp.float32)
    z = z + b_ref[...]
    h0 = jnp.maximum(z * g_ref[...] + bt_ref[...], 0.0)
    deg = d0_ref[0] + d1_ref[0] + 1.0
    dinv = lax.rsqrt(deg)
    h0_ref[...] = h0
    dinv_ref[...] = dinv
    hs_ref[...] = h0 * dinv


def _tc_layer(bl, p0_ref, p1_ref, hs_ref, h0_ref, dinv_ref, w_ref,
              g_ref, bt_ref, out_ref):
    dinv = dinv_ref[...]
    t = dinv * (p0_ref[0] + p1_ref[0] + hs_ref[...])
    sup = (1.0 - _ALPHA) * t + _ALPHA * h0_ref[...]
    z = (1.0 - bl) * sup + bl * jnp.dot(
        sup, w_ref[...], preferred_element_type=jnp.float32)
    h = jnp.maximum(z * g_ref[...] + bt_ref[...], 0.0)
    out_ref[...] = h * dinv


def _tc_layer_last(bl, p0_ref, p1_ref, hs_ref, h0_ref, dinv_ref, w_ref,
                   g_ref, bt_ref, wo_ref, bo_ref, out_ref):
    dinv = dinv_ref[...]
    t = dinv * (p0_ref[0] + p1_ref[0] + hs_ref[...])
    sup = (1.0 - _ALPHA) * t + _ALPHA * h0_ref[...]
    z = (1.0 - bl) * sup + bl * jnp.dot(
        sup, w_ref[...], preferred_element_type=jnp.float32)
    h = jnp.maximum(z * g_ref[...] + bt_ref[...], 0.0)
    out_ref[...] = jnp.dot(
        h, wo_ref[...], preferred_element_type=jnp.float32) + bo_ref[...]


_row = pl.BlockSpec((_BLK, _NHID), lambda i: (i, 0))
_full = pl.BlockSpec((_NHID, _NHID), lambda i: (0, 0))
_vec = pl.BlockSpec((1, _NHID), lambda i: (0, 0))
_dcol = pl.BlockSpec((_BLK, 1), lambda i: (i, 0))
_p0 = pl.BlockSpec((1, _BLK, _NHID), lambda i: (0, i, 0))
_p1 = pl.BlockSpec((1, _BLK, _NHID), lambda i: (1, i, 0))
_d0 = pl.BlockSpec((1, _BLK, 1), lambda i: (0, i, 0))
_d1 = pl.BlockSpec((1, _BLK, 1), lambda i: (1, i, 0))

_f32 = jnp.float32


def kernel(x, edge_index, fc_in_w, fc_in_b, bn_in_gamma, bn_in_beta,
           conv_w, bn_gamma, bn_beta, fc_out_w, fc_out_b):
    src = edge_index[0]
    dst = edge_index[1]
    pad = _EPAD - src.shape[0]
    src3 = jnp.concatenate(
        [src, jnp.zeros((pad,), jnp.int32)]).reshape(_NW, _CPW, _CH)
    dst3 = jnp.concatenate(
        [dst, jnp.full((pad,), _N, jnp.int32)]).reshape(_NW, _CPW, _CH)

    mesh = plsc.VectorSubcoreMesh(core_axis_name="c", subcore_axis_name="s")

    degP = pl.kernel(
        _sc_deg,
        out_type=jax.ShapeDtypeStruct((_NC, _DN), _f32),
        mesh=mesh,
        scratch_types=[
            pltpu.VMEM((_CPW, _CH), jnp.int32),
            pltpu.VMEM((_CH,), _f32),
            pltpu.VMEM((_DN,), _f32),
            pltpu.SemaphoreType.DMA,
            pltpu.VMEM_SHARED((_DN,), _f32),
        ],
    )(dst3)
    degP3 = degP.reshape(_NC, _DN, 1)

    bn_scale = 1.0 / math.sqrt(1.0 + _EPS)
    gin = (bn_in_gamma * bn_scale).reshape(1, _NHID)
    g_all = bn_gamma * bn_scale

    h0, hs, dinv = pl.pallas_call(
        _tc_in,
        grid=(_GRID,),
        in_specs=[_row, _full, _vec, _vec, _vec, _d0, _d1],
        out_specs=[_row, _row, _dcol],
        out_shape=[
            jax.ShapeDtypeStruct((_N, _NHID), _f32),
            jax.ShapeDtypeStruct((_N, _NHID), _f32),
            jax.ShapeDtypeStruct((_N, 1), _f32),
        ],
    )(x, fc_in_w, fc_in_b.reshape(1, _NHID), gin,
      bn_in_beta.reshape(1, _NHID), degP3, degP3)

    prop = pl.kernel(
        _sc_prop,
        out_type=jax.ShapeDtypeStruct((_NC, _ND, _NHID), _f32),
        mesh=mesh,
        scratch_types=(
            [pltpu.VMEM((_CPG, _CH), jnp.int32)] * 2
            + [pltpu.VMEM((_CH, _NHID), _f32)] * _NBUF
            + [pltpu.SemaphoreType.DMA] * (2 * _NBUF)
            + [pltpu.VMEM_SHARED((_ND, _NHID), _f32)]
        ),
    )

    wo = jnp.zeros((_NHID, _NHID), _f32).at[:, :_NCLASS].set(fc_out_w)
    bo = jnp.zeros((1, _NHID), _f32).at[0, :_NCLASS].set(fc_out_b)

    for i in range(_L):
        P = prop(hs, src3, dst3)
        bl = math.log(_THETA / (i + 1) + 1.0)
        gi = g_all[i].reshape(1, _NHID)
        bti = bn_beta[i].reshape(1, _NHID)
        if i < _L - 1:
            hs = pl.pallas_call(
                functools.partial(_tc_layer, bl),
                grid=(_GRID,),
                in_specs=[_p0, _p1, _row, _row, _dcol, _full, _vec, _vec],
                out_specs=_row,
                out_shape=jax.ShapeDtypeStruct((_N, _NHID), _f32),
            )(P, P, hs, h0, dinv, conv_w[i], gi, bti)
        else:
            outp = pl.pallas_call(
                functools.partial(_tc_layer_last, bl),
                grid=(_GRID,),
                in_specs=[_p0, _p1, _row, _row, _dcol, _full, _vec, _vec,
                          _full, _vec],
                out_specs=_row,
                out_shape=jax.ShapeDtypeStruct((_N, _NHID), _f32),
            )(P, P, hs, h0, dinv, conv_w[i], gi, bti, wo, bo)

    return outp[:, :_NCLASS]


# speedup vs baseline: 1.0313x; 1.0313x over previous
import functools
import math

import jax
import jax.numpy as jnp
from jax import lax
from jax.experimental import pallas as pl
from jax.experimental.pallas import tpu as pltpu
from jax.experimental.pallas import tpu_sc as plsc

_N = 10000
_NHID = 128
_NCLASS = 40
_L = 8
_ALPHA = 0.1
_THETA = 0.5
_EPS = 1e-5

_NC = 2
_NS = 16
_NW = _NC * _NS
_CH = 128
_CPW = 80
_EPAD = _NW * _CPW * _CH
_ND = 10112
_RPT = _ND // _NS
_ZCH = (128, 128, 128, 128, 120)
_CPG = 8
_DN = 10240

_BLK = 400
_GRID = _N // _BLK


def _sc_deg(dst3, out, dst_v, ones_v, zbuf, sem, deg_sp):
    cid = lax.axis_index("c")
    sid = lax.axis_index("s")
    wid = cid * _NS + sid
    for c in range(_CH // 16):
        ones_v[pl.ds(c * 16, 16)] = jnp.ones((16,), jnp.float32)

    @pl.when(sid == 0)
    def _zero():
        def zrow(r, carry):
            zbuf[pl.ds(r * 16, 16)] = jnp.zeros((16,), jnp.float32)
            return carry
        lax.fori_loop(0, _DN // 16, zrow, 0)
        pltpu.sync_copy(zbuf, deg_sp)

    pltpu.sync_copy(dst3.at[wid], dst_v)
    plsc.subcore_barrier()

    def group(g, carry):
        for j in range(_CPG):
            pltpu.async_copy(ones_v, deg_sp.at[dst_v.at[g * _CPG + j]],
                             sem, add=True)
        for j in range(_CPG):
            pltpu.make_async_copy(
                ones_v, deg_sp.at[dst_v.at[g * _CPG + j]], sem).wait()
        return carry
    lax.fori_loop(0, _CPW // _CPG, group, 0)
    plsc.subcore_barrier()

    @pl.when(sid == 0)
    def _copy_out():
        pltpu.sync_copy(deg_sp, out.at[cid])


def _sc_prop(hs, src3, dst3, out, src_v, dst_v, rows0, rows1, sem0, sem1,
             agg):
    cid = lax.axis_index("c")
    sid = lax.axis_index("s")
    wid = cid * _NS + sid

    def zrow(r, carry):
        for c in range(_NHID // 16):
            rows0[r, pl.ds(c * 16, 16)] = jnp.zeros((16,), jnp.float32)
        return carry
    lax.fori_loop(0, _CH, zrow, 0)
    base = sid * _RPT
    off = 0
    for zc in _ZCH:
        pltpu.sync_copy(rows0.at[pl.ds(0, zc)],
                        agg.at[pl.ds(base + off, zc)])
        off += zc
    plsc.subcore_barrier()

    rows = (rows0, rows1)
    sems = (sem0, sem1)

    def group(g, carry):
        goff = pl.multiple_of(g * _CPG, _CPG)
        pltpu.sync_copy(src3.at[wid, pl.ds(goff, _CPG)], src_v)
        pltpu.sync_copy(dst3.at[wid, pl.ds(goff, _CPG)], dst_v)
        cps = [pltpu.async_copy(hs.at[src_v.at[0]], rows[0], sems[0])]
        for j in range(_CPG):
            if j + 1 < _CPG:
                cps.append(pltpu.async_copy(
                    hs.at[src_v.at[j + 1]], rows[(j + 1) % 2],
                    sems[(j + 1) % 2]))
            cps[j].wait()
            pltpu.sync_copy(rows[j % 2], agg.at[dst_v.at[j]], add=True)
        return carry
    lax.fori_loop(0, _CPW // _CPG, group, 0)
    plsc.subcore_barrier()
    pltpu.sync_copy(agg.at[pl.ds(base, _RPT)], out.at[cid, pl.ds(base, _RPT)])


def _tc_in(x_ref, w_ref, b_ref, g_ref, bt_ref, d0_ref, d1_ref,
           h0_ref, hs_ref, dinv_ref):
    z = jnp.dot(x_ref[...], w_ref[...], preferred_element_type=jnp.float32)
    z = z + b_ref[...]
    h0 = jnp.maximum(z * g_ref[...] + bt_ref[...], 0.0)
    deg = d0_ref[0] + d1_ref[0] + 1.0
    dinv = lax.rsqrt(deg)
    h0_ref[...] = h0
    dinv_ref[...] = dinv
    hs_ref[...] = h0 * dinv


def _tc_layer(bl, p0_ref, p1_ref, hs_ref, h0_ref, dinv_ref, w_ref,
              g_ref, bt_ref, out_ref):
    dinv = dinv_ref[...]
    t = dinv * (p0_ref[0] + p1_ref[0] + hs_ref[...])
    sup = (1.0 - _ALPHA) * t + _ALPHA * h0_ref[...]
    z = (1.0 - bl) * sup + bl * jnp.dot(
        sup, w_ref[...], preferred_element_type=jnp.float32)
    h = jnp.maximum(z * g_ref[...] + bt_ref[...], 0.0)
    out_ref[...] = h * dinv


def _tc_layer_last(bl, p0_ref, p1_ref, hs_ref, h0_ref, dinv_ref, w_ref,
                   g_ref, bt_ref, wo_ref, bo_ref, out_ref):
    dinv = dinv_ref[...]
    t = dinv * (p0_ref[0] + p1_ref[0] + hs_ref[...])
    sup = (1.0 - _ALPHA) * t + _ALPHA * h0_ref[...]
    z = (1.0 - bl) * sup + bl * jnp.dot(
        sup, w_ref[...], preferred_element_type=jnp.float32)
    h = jnp.maximum(z * g_ref[...] + bt_ref[...], 0.0)
    out_ref[...] = jnp.dot(
        h, wo_ref[...], preferred_element_type=jnp.float32) + bo_ref[...]


_row = pl.BlockSpec((_BLK, _NHID), lambda i: (i, 0))
_full = pl.BlockSpec((_NHID, _NHID), lambda i: (0, 0))
_vec = pl.BlockSpec((1, _NHID), lambda i: (0, 0))
_dcol = pl.BlockSpec((_BLK, 1), lambda i: (i, 0))
_p0 = pl.BlockSpec((1, _BLK, _NHID), lambda i: (0, i, 0))
_p1 = pl.BlockSpec((1, _BLK, _NHID), lambda i: (1, i, 0))
_d0 = pl.BlockSpec((1, _BLK, 1), lambda i: (0, i, 0))
_d1 = pl.BlockSpec((1, _BLK, 1), lambda i: (1, i, 0))

_f32 = jnp.float32


def kernel(x, edge_index, fc_in_w, fc_in_b, bn_in_gamma, bn_in_beta,
           conv_w, bn_gamma, bn_beta, fc_out_w, fc_out_b):
    src = edge_index[0]
    dst = edge_index[1]
    pad = _EPAD - src.shape[0]
    src3 = jnp.concatenate(
        [src, jnp.zeros((pad,), jnp.int32)]).reshape(_NW, _CPW, _CH)
    dst3 = jnp.concatenate(
        [dst, jnp.full((pad,), _N, jnp.int32)]).reshape(_NW, _CPW, _CH)

    mesh = plsc.VectorSubcoreMesh(core_axis_name="c", subcore_axis_name="s")

    degP = pl.kernel(
        _sc_deg,
        out_type=jax.ShapeDtypeStruct((_NC, _DN), _f32),
        mesh=mesh,
        scratch_types=[
            pltpu.VMEM((_CPW, _CH), jnp.int32),
            pltpu.VMEM((_CH,), _f32),
            pltpu.VMEM((_DN,), _f32),
            pltpu.SemaphoreType.DMA,
            pltpu.VMEM_SHARED((_DN,), _f32),
        ],
    )(dst3)
    degP3 = degP.reshape(_NC, _DN, 1)

    bn_scale = 1.0 / math.sqrt(1.0 + _EPS)
    gin = (bn_in_gamma * bn_scale).reshape(1, _NHID)
    g_all = bn_gamma * bn_scale

    h0, hs, dinv = pl.pallas_call(
        _tc_in,
        grid=(_GRID,),
        in_specs=[_row, _full, _vec, _vec, _vec, _d0, _d1],
        out_specs=[_row, _row, _dcol],
        out_shape=[
            jax.ShapeDtypeStruct((_N, _NHID), _f32),
            jax.ShapeDtypeStruct((_N, _NHID), _f32),
            jax.ShapeDtypeStruct((_N, 1), _f32),
        ],
    )(x, fc_in_w, fc_in_b.reshape(1, _NHID), gin,
      bn_in_beta.reshape(1, _NHID), degP3, degP3)

    prop = pl.kernel(
        _sc_prop,
        out_type=jax.ShapeDtypeStruct((_NC, _ND, _NHID), _f32),
        mesh=mesh,
        scratch_types=[
            pltpu.VMEM((_CPG, _CH), jnp.int32),
            pltpu.VMEM((_CPG, _CH), jnp.int32),
            pltpu.VMEM((_CH, _NHID), _f32),
            pltpu.VMEM((_CH, _NHID), _f32),
            pltpu.SemaphoreType.DMA,
            pltpu.SemaphoreType.DMA,
            pltpu.VMEM_SHARED((_ND, _NHID), _f32),
        ],
    )

    wo = jnp.zeros((_NHID, _NHID), _f32).at[:, :_NCLASS].set(fc_out_w)
    bo = jnp.zeros((1, _NHID), _f32).at[0, :_NCLASS].set(fc_out_b)

    for i in range(_L):
        P = prop(hs, src3, dst3)
        bl = math.log(_THETA / (i + 1) + 1.0)
        gi = g_all[i].reshape(1, _NHID)
        bti = bn_beta[i].reshape(1, _NHID)
        if i < _L - 1:
            hs = pl.pallas_call(
                functools.partial(_tc_layer, bl),
                grid=(_GRID,),
                in_specs=[_p0, _p1, _row, _row, _dcol, _full, _vec, _vec],
                out_specs=_row,
                out_shape=jax.ShapeDtypeStruct((_N, _NHID), _f32),
            )(P, P, hs, h0, dinv, conv_w[i], gi, bti)
        else:
            outp = pl.pallas_call(
                functools.partial(_tc_layer_last, bl),
                grid=(_GRID,),
                in_specs=[_p0, _p1, _row, _row, _dcol, _full, _vec, _vec,
                          _full, _vec],
                out_specs=_row,
                out_shape=jax.ShapeDtypeStruct((_N, _NHID), _f32),
            )(P, P, hs, h0, dinv, conv_w[i], gi, bti, wo, bo)

    return outp[:, :_NCLASS]
